# initial kernel scaffold (unmeasured)
import jax
import jax.numpy as jnp
from jax import lax
from jax.experimental import pallas as pl
from jax.experimental.pallas import tpu as pltpu

N_DEV = 16


def kernel(x, w_mat, scale_x, scale_w):
    M_global, k_per = x.shape
    K, n = w_mat.shape
    m_per = M_global // N_DEV

    def body(x_ref, w_ref, sx_ref, sw_ref, out_ref,
             blocks, send_sems, recv_sems):
        my = lax.axis_index("i")

        blocks[0] = x_ref[pl.ds(my * m_per, m_per), :]

        rdmas = []
        for d in range(1, N_DEV):
            j = lax.rem(my + d, N_DEV)
            rdma = pltpu.make_async_remote_copy(
                src_ref=x_ref.at[pl.ds(j * m_per, m_per), :],
                dst_ref=blocks.at[d],
                send_sem=send_sems.at[d],
                recv_sem=recv_sems.at[d],
                device_id=(j,),
                device_id_type=pl.DeviceIdType.MESH,
            )
            rdma.start()
            rdmas.append(rdma)

        for d in range(N_DEV):
            if d > 0:
                rdmas[d - 1].wait_recv()
            k_idx = lax.rem(my - d + N_DEV, N_DEV)
            wslice = w_ref[pl.ds(k_idx * k_per, k_per), :]
            partial = lax.dot_general(
                blocks[d], wslice,
                (((1,), (0,)), ((), ())),
                preferred_element_type=jnp.float32,
            )
            if d == 0:
                out_ref[...] = partial
            else:
                out_ref[...] += partial

        for r in rdmas:
            r.wait_send()

        s = sx_ref[0] * sw_ref[0]
        y = out_ref[...] * s
        z = jnp.clip(y, -60.0, 60.0)
        out_ref[...] = y / (1.0 + jnp.exp(-z))

    return pl.pallas_call(
        body,
        out_shape=jax.ShapeDtypeStruct((m_per, n), jnp.float32),
        in_specs=[
            pl.BlockSpec(memory_space=pltpu.VMEM),
            pl.BlockSpec(memory_space=pltpu.VMEM),
            pl.BlockSpec(memory_space=pltpu.SMEM),
            pl.BlockSpec(memory_space=pltpu.SMEM),
        ],
        out_specs=pl.BlockSpec(memory_space=pltpu.VMEM),
        scratch_shapes=[
            pltpu.VMEM((N_DEV, m_per, k_per), x.dtype),
            pltpu.SemaphoreType.DMA((N_DEV,)),
            pltpu.SemaphoreType.DMA((N_DEV,)),
        ],
        compiler_params=pltpu.CompilerParams(
            collective_id=0,
            vmem_limit_bytes=100 * 1024 * 1024,
        ),
    )(x, w_mat, scale_x, scale_w)


# baseline (device time: 71464 ns/iter reference)
import jax
import jax.numpy as jnp
from jax import lax
from jax.experimental import pallas as pl
from jax.experimental.pallas import tpu as pltpu

N_DEV = 16
N_TILE = 1024


def kernel(x, w_mat, scale_x, scale_w):
    M_global, k_per = x.shape
    K, n = w_mat.shape
    m_per = M_global // N_DEV
    n_tiles = n // N_TILE

    def body(x_ref, w_ref, sx_ref, sw_ref, out_ref,
             x8, xfull, xbf, wtile, send_sems, recv_sems, wsems):
        my = lax.axis_index("i")

        x8[...] = x_ref[...].astype(jnp.float8_e4m3fn)

        wdma0 = pltpu.make_async_copy(
            w_ref.at[:, pl.ds(0, N_TILE)], wtile.at[0], wsems.at[0]
        )
        wdma0.start()

        rdmas = []
        for d in range(1, N_DEV):
            j = lax.rem(my + d, N_DEV)
            rdma = pltpu.make_async_remote_copy(
                src_ref=x8.at[pl.ds(j * m_per, m_per), :],
                dst_ref=xfull.at[:, pl.ds(my * k_per, k_per)],
                send_sem=send_sems.at[d],
                recv_sem=recv_sems.at[d],
                device_id=(j,),
                device_id_type=pl.DeviceIdType.MESH,
            )
            rdma.start()
            rdmas.append(rdma)

        xfull[:, pl.ds(my * k_per, k_per)] = x8[pl.ds(my * m_per, m_per), :]

        for r in rdmas:
            r.wait_recv()
        xbf[...] = xfull[...].astype(jnp.bfloat16)

        s = sx_ref[0] * sw_ref[0]
        for t in range(n_tiles):
            slot = t % 2
            if t + 1 < n_tiles:
                nxt = pltpu.make_async_copy(
                    w_ref.at[:, pl.ds((t + 1) * N_TILE, N_TILE)],
                    wtile.at[(t + 1) % 2],
                    wsems.at[(t + 1) % 2],
                )
                nxt.start()
            pltpu.make_async_copy(
                w_ref.at[:, pl.ds(t * N_TILE, N_TILE)],
                wtile.at[slot],
                wsems.at[slot],
            ).wait()
            acc = lax.dot_general(
                xbf[...], wtile[slot].astype(jnp.bfloat16),
                (((1,), (0,)), ((), ())),
                preferred_element_type=jnp.float32,
            )
            y = acc * s
            z = jnp.clip(y, -60.0, 60.0)
            out_ref[:, pl.ds(t * N_TILE, N_TILE)] = y / (1.0 + jnp.exp(-z))

        for r in rdmas:
            r.wait_send()

    return pl.pallas_call(
        body,
        out_shape=jax.ShapeDtypeStruct((m_per, n), jnp.float32),
        in_specs=[
            pl.BlockSpec(memory_space=pltpu.VMEM),
            pl.BlockSpec(memory_space=pltpu.MemorySpace.HBM),
            pl.BlockSpec(memory_space=pltpu.SMEM),
            pl.BlockSpec(memory_space=pltpu.SMEM),
        ],
        out_specs=pl.BlockSpec(memory_space=pltpu.VMEM),
        scratch_shapes=[
            pltpu.VMEM((M_global, k_per), jnp.float8_e4m3fn),
            pltpu.VMEM((m_per, K), jnp.float8_e4m3fn),
            pltpu.VMEM((m_per, K), jnp.bfloat16),
            pltpu.VMEM((2, K, N_TILE), jnp.float32),
            pltpu.SemaphoreType.DMA((N_DEV,)),
            pltpu.SemaphoreType.DMA((N_DEV,)),
            pltpu.SemaphoreType.DMA((2,)),
        ],
        compiler_params=pltpu.CompilerParams(
            vmem_limit_bytes=100 * 1024 * 1024,
        ),
    )(x, w_mat, scale_x, scale_w)


# device time: 66792 ns/iter; 1.0699x vs baseline; 1.0699x over previous
import jax
import jax.numpy as jnp
from jax import lax
from jax.experimental import pallas as pl
from jax.experimental.pallas import tpu as pltpu

N_DEV = 16
N_TILE = 1024


def kernel(x, w_mat, scale_x, scale_w):
    M_global, k_per = x.shape
    K, n = w_mat.shape
    m_per = M_global // N_DEV
    n_tiles = n // N_TILE

    def body(x_ref, w_ref, sx_ref, sw_ref, out_ref,
             x8, xfull, wtile, send_sems, recv_sems, wsems):
        my = lax.axis_index("i")

        x8[...] = x_ref[...].astype(jnp.float8_e4m3fn)

        for t0 in range(2):
            pltpu.make_async_copy(
                w_ref.at[:, pl.ds(t0 * N_TILE, N_TILE)],
                wtile.at[t0], wsems.at[t0],
            ).start()

        rdmas = []
        for d in range(1, N_DEV):
            j = lax.rem(my + d, N_DEV)
            rdma = pltpu.make_async_remote_copy(
                src_ref=x8.at[pl.ds(j * m_per, m_per), :],
                dst_ref=xfull.at[:, pl.ds(my * k_per, k_per)],
                send_sem=send_sems.at[d],
                recv_sem=recv_sems.at[d],
                device_id=(j,),
                device_id_type=pl.DeviceIdType.MESH,
            )
            rdma.start()
            rdmas.append(rdma)

        xfull[:, pl.ds(my * k_per, k_per)] = x8[pl.ds(my * m_per, m_per), :]

        for r in rdmas:
            r.wait_recv()

        s = sx_ref[0] * sw_ref[0]
        for t in range(n_tiles):
            slot = t % 2
            pltpu.make_async_copy(
                w_ref.at[:, pl.ds(t * N_TILE, N_TILE)],
                wtile.at[slot],
                wsems.at[slot],
            ).wait()
            acc = lax.dot_general(
                xfull[...], wtile[slot].astype(jnp.float8_e5m2),
                (((1,), (0,)), ((), ())),
                preferred_element_type=jnp.float32,
            )
            y = acc * s
            z = jnp.clip(y, -60.0, 60.0)
            out_ref[:, pl.ds(t * N_TILE, N_TILE)] = y / (1.0 + jnp.exp(-z))
            if t + 2 < n_tiles:
                pltpu.make_async_copy(
                    w_ref.at[:, pl.ds((t + 2) * N_TILE, N_TILE)],
                    wtile.at[slot],
                    wsems.at[slot],
                ).start()

        for r in rdmas:
            r.wait_send()

    return pl.pallas_call(
        body,
        out_shape=jax.ShapeDtypeStruct((m_per, n), jnp.float32),
        in_specs=[
            pl.BlockSpec(memory_space=pltpu.VMEM),
            pl.BlockSpec(memory_space=pltpu.MemorySpace.HBM),
            pl.BlockSpec(memory_space=pltpu.SMEM),
            pl.BlockSpec(memory_space=pltpu.SMEM),
        ],
        out_specs=pl.BlockSpec(memory_space=pltpu.VMEM),
        scratch_shapes=[
            pltpu.VMEM((M_global, k_per), jnp.float8_e4m3fn),
            pltpu.VMEM((m_per, K), jnp.float8_e4m3fn),
            pltpu.VMEM((2, K, N_TILE), jnp.float32),
            pltpu.SemaphoreType.DMA((N_DEV,)),
            pltpu.SemaphoreType.DMA((N_DEV,)),
            pltpu.SemaphoreType.DMA((2,)),
        ],
        compiler_params=pltpu.CompilerParams(
            vmem_limit_bytes=100 * 1024 * 1024,
        ),
    )(x, w_mat, scale_x, scale_w)
